# baseline (speedup 1.0000x reference)
import functools

import jax
import jax.numpy as jnp
from jax import lax
from jax.experimental import pallas as pl
from jax.experimental.pallas import tpu as pltpu
from jax.experimental.pallas import tpu_sc as plsc

_NQ = 8
_K = 1024
_D = 64
_BLK = 512

_NC = 2
_NS = 16
_NW = _NC * _NS
_ROWS = 8192 * _NQ
_RPW = _ROWS // _NW
_CH = 512
_IPD = 128
_DP = 128
_PAIR_UNROLL = 4


def _score_body(x_ref, cbt_ref, ids_ref, gidx_ref):
    ids_cols = []
    gidx_cols = []
    for q in range(_NQ):
        xq = x_ref[:, q * _D:(q + 1) * _D]
        cbt = cbt_ref[q]
        cnorm = jnp.sum(cbt * cbt, axis=0, keepdims=True)
        scores = cnorm - jnp.dot(
            xq + xq, cbt, preferred_element_type=jnp.float32)
        ids = jnp.argmin(scores, axis=1).astype(jnp.int32)
        ids_cols.append(ids[:, None])
        gidx_cols.append(ids[:, None] + q * _K)
    ids_ref[...] = jnp.concatenate(ids_cols, axis=1)
    gidx_ref[...] = jnp.concatenate(gidx_cols, axis=1)


def _sc_gather_body(table_ref, gidx_ref, out_ref, idx_v, rows_v, packed_v, sem):
    wid = lax.axis_index("s") * _NC + lax.axis_index("c")
    base = wid * _RPW
    pltpu.sync_copy(gidx_ref.at[pl.ds(base, _RPW)], idx_v)
    for c in range(_RPW // _CH):
        cps = []
        for j in range(_CH // _IPD):
            off = c * _CH + j * _IPD
            cps.append(pltpu.async_copy(
                table_ref.at[idx_v.at[pl.ds(off, _IPD)]],
                rows_v.at[pl.ds(j * _IPD, _IPD)], sem))
        for cp in cps:
            cp.wait()

        def _pack(i, carry):
            for u in range(_PAIR_UNROLL):
                p = i * _PAIR_UNROLL + u
                r0 = rows_v.at[2 * p]
                r1 = rows_v.at[2 * p + 1]
                dst = packed_v.at[p]
                for k in range(_D // 16):
                    dst[pl.ds(k * 16, 16)] = r0[pl.ds(k * 16, 16)]
                    dst[pl.ds(_D + k * 16, 16)] = r1[pl.ds(k * 16, 16)]
            return carry

        lax.fori_loop(0, (_CH // 2) // _PAIR_UNROLL, _pack, 0)
        pltpu.sync_copy(
            packed_v,
            out_ref.at[pl.ds(pl.multiple_of((base + c * _CH) // 2, 8),
                             _CH // 2)])


def kernel(x, codebooks):
    B = x.shape[0]
    cbt = codebooks.transpose(0, 2, 1)

    ids_bq, gidx_bq = pl.pallas_call(
        _score_body,
        grid=(B // _BLK,),
        in_specs=[
            pl.BlockSpec((_BLK, _NQ * _D), lambda i: (i, 0)),
            pl.BlockSpec((_NQ, _D, _K), lambda i: (0, 0, 0)),
        ],
        out_specs=[
            pl.BlockSpec((_BLK, _NQ), lambda i: (i, 0)),
            pl.BlockSpec((_BLK, _NQ), lambda i: (i, 0)),
        ],
        out_shape=[
            jax.ShapeDtypeStruct((B, _NQ), jnp.int32),
            jax.ShapeDtypeStruct((B, _NQ), jnp.int32),
        ],
    )(x, cbt)

    table = jnp.pad(codebooks.reshape(_NQ * _K, _D),
                    ((0, 0), (0, _DP - _D)))
    gidx_flat = gidx_bq.reshape(B * _NQ)

    sc_gather = functools.partial(
        pl.kernel,
        mesh=plsc.VectorSubcoreMesh(core_axis_name="c", subcore_axis_name="s"),
        out_type=jax.ShapeDtypeStruct((_ROWS // 2, _DP), jnp.float32),
        scratch_types=[
            pltpu.VMEM((_RPW,), jnp.int32),
            pltpu.VMEM((_CH, _DP), jnp.float32),
            pltpu.VMEM((_CH // 2, _DP), jnp.float32),
            pltpu.SemaphoreType.DMA,
        ],
    )(_sc_gather_body)

    q_rows = sc_gather(table, gidx_flat)
    return (q_rows.reshape(B, _NQ * _D),
            ids_bq.T.astype(jnp.int64))


# speedup vs baseline: 1.0244x; 1.0244x over previous
import functools

import jax
import jax.numpy as jnp
from jax import lax
from jax.experimental import pallas as pl
from jax.experimental.pallas import tpu as pltpu
from jax.experimental.pallas import tpu_sc as plsc

_NQ = 8
_K = 1024
_D = 64
_BLK = 512

_NC = 2
_NS = 16
_NW = _NC * _NS
_CH = 512
_IPD = 128
_DP = 128
_PAIR_UNROLL = 4


def _score_body(x_ref, cbt_ref, ids_ref, gidx_ref):
    ids_cols = []
    gidx_cols = []
    for q in range(_NQ):
        xq = x_ref[:, q * _D:(q + 1) * _D]
        cbt = cbt_ref[q]
        cnorm = jnp.sum(cbt * cbt, axis=0, keepdims=True)
        scores = cnorm - jnp.dot(
            xq + xq, cbt, preferred_element_type=jnp.float32)
        ids = jnp.argmin(scores, axis=1).astype(jnp.int32)
        ids_cols.append(ids[:, None])
        gidx_cols.append(ids[:, None] + q * _K)
    ids_ref[...] = jnp.concatenate(ids_cols, axis=1)
    gidx_ref[...] = jnp.concatenate(gidx_cols, axis=1)


def _make_sc_gather(n_rows):
    rpw = n_rows // _NW

    def _sc_gather_body(table_ref, gidx_ref, out_ref, idx_v, rows_v,
                        packed_v, sem):
        wid = lax.axis_index("s") * _NC + lax.axis_index("c")
        base = wid * rpw
        pltpu.sync_copy(gidx_ref.at[pl.ds(base, rpw)], idx_v)
        for c in range(rpw // _CH):
            cps = []
            for j in range(_CH // _IPD):
                off = c * _CH + j * _IPD
                cps.append(pltpu.async_copy(
                    table_ref.at[idx_v.at[pl.ds(off, _IPD)]],
                    rows_v.at[pl.ds(j * _IPD, _IPD)], sem))
            for cp in cps:
                cp.wait()

            def _pack(i, carry):
                for u in range(_PAIR_UNROLL):
                    p = i * _PAIR_UNROLL + u
                    r0 = rows_v.at[2 * p]
                    r1 = rows_v.at[2 * p + 1]
                    dst = packed_v.at[p]
                    for k in range(_D // 16):
                        dst[pl.ds(k * 16, 16)] = r0[pl.ds(k * 16, 16)]
                        dst[pl.ds(_D + k * 16, 16)] = r1[pl.ds(k * 16, 16)]
                return carry

            lax.fori_loop(0, (_CH // 2) // _PAIR_UNROLL, _pack, 0)
            pltpu.sync_copy(
                packed_v,
                out_ref.at[pl.ds(pl.multiple_of((base + c * _CH) // 2, 8),
                                 _CH // 2)])

    return functools.partial(
        pl.kernel,
        mesh=plsc.VectorSubcoreMesh(core_axis_name="c", subcore_axis_name="s"),
        out_type=jax.ShapeDtypeStruct((n_rows // 2, _DP), jnp.float32),
        scratch_types=[
            pltpu.VMEM((rpw,), jnp.int32),
            pltpu.VMEM((_CH, _DP), jnp.float32),
            pltpu.VMEM((_CH // 2, _DP), jnp.float32),
            pltpu.SemaphoreType.DMA,
        ],
    )(_sc_gather_body)


def _score_call(xh, cbt):
    bh = xh.shape[0]
    return pl.pallas_call(
        _score_body,
        grid=(bh // _BLK,),
        in_specs=[
            pl.BlockSpec((_BLK, _NQ * _D), lambda i: (i, 0)),
            pl.BlockSpec((_NQ, _D, _K), lambda i: (0, 0, 0)),
        ],
        out_specs=[
            pl.BlockSpec((_BLK, _NQ), lambda i: (i, 0)),
            pl.BlockSpec((_BLK, _NQ), lambda i: (i, 0)),
        ],
        out_shape=[
            jax.ShapeDtypeStruct((bh, _NQ), jnp.int32),
            jax.ShapeDtypeStruct((bh, _NQ), jnp.int32),
        ],
    )(xh, cbt)


def kernel(x, codebooks):
    B = x.shape[0]
    half = B // 2
    cbt = codebooks.transpose(0, 2, 1)
    table = jnp.pad(codebooks.reshape(_NQ * _K, _D),
                    ((0, 0), (0, _DP - _D)))
    sc_gather = _make_sc_gather(half * _NQ)

    ids_a, gidx_a = _score_call(x[:half], cbt)
    ids_b, gidx_b = _score_call(x[half:], cbt)
    qa = sc_gather(table, gidx_a.reshape(half * _NQ))
    qb = sc_gather(table, gidx_b.reshape(half * _NQ))

    q_out = jnp.concatenate(
        [qa.reshape(half, _NQ * _D), qb.reshape(half, _NQ * _D)], axis=0)
    ids = jnp.concatenate([ids_a, ids_b], axis=0)
    return q_out, ids.T.astype(jnp.int64)


# speedup vs baseline: 1.0442x; 1.0194x over previous
import functools

import jax
import jax.numpy as jnp
from jax import lax
from jax.experimental import pallas as pl
from jax.experimental.pallas import tpu as pltpu
from jax.experimental.pallas import tpu_sc as plsc

_NQ = 8
_K = 1024
_D = 64
_BLK = 512

_NC = 2
_NS = 16
_NW = _NC * _NS
_CH = 256
_IPD = 128
_DP = 128
_PAIR_UNROLL = 4


def _score_body(x_ref, cbt_ref, ids_ref, gidx_ref):
    ids_cols = []
    gidx_cols = []
    for q in range(_NQ):
        xq = x_ref[:, q * _D:(q + 1) * _D]
        cbt = cbt_ref[q]
        cnorm = jnp.sum(cbt * cbt, axis=0, keepdims=True)
        scores = cnorm - jnp.dot(
            xq + xq, cbt, preferred_element_type=jnp.float32)
        ids = jnp.argmin(scores, axis=1).astype(jnp.int32)
        ids_cols.append(ids[:, None])
        gidx_cols.append(ids[:, None] + q * _K)
    ids_ref[...] = jnp.concatenate(ids_cols, axis=1)
    gidx_ref[...] = jnp.concatenate(gidx_cols, axis=1)


def _make_sc_gather(n_rows):
    rpw = n_rows // _NW
    nch = rpw // _CH

    def _sc_gather_body(table_ref, gidx_ref, out_ref, idx_v,
                        rows0, rows1, pk0, pk1, sg0, sg1, so0, so1):
        wid = lax.axis_index("s") * _NC + lax.axis_index("c")
        base = wid * rpw
        rows = (rows0, rows1)
        pk = (pk0, pk1)
        sg = (sg0, sg1)
        so = (so0, so1)
        pltpu.sync_copy(gidx_ref.at[pl.ds(base, rpw)], idx_v)

        def _fire(c):
            buf = rows[c % 2]
            return [pltpu.async_copy(
                table_ref.at[idx_v.at[pl.ds(c * _CH + j * _IPD, _IPD)]],
                buf.at[pl.ds(j * _IPD, _IPD)], sg[c % 2])
                for j in range(_CH // _IPD)]

        def _pack_chunk(src, dst):
            def _pack(i, carry):
                for u in range(_PAIR_UNROLL):
                    p = i * _PAIR_UNROLL + u
                    r0 = src.at[2 * p]
                    r1 = src.at[2 * p + 1]
                    d = dst.at[p]
                    for k in range(_D // 16):
                        d[pl.ds(k * 16, 16)] = r0[pl.ds(k * 16, 16)]
                        d[pl.ds(_D + k * 16, 16)] = r1[pl.ds(k * 16, 16)]
                return carry
            lax.fori_loop(0, (_CH // 2) // _PAIR_UNROLL, _pack, 0)

        pending = {0: _fire(0)}
        out_cps = {}
        for c in range(nch):
            if c + 1 < nch:
                pending[c + 1] = _fire(c + 1)
            for cp in pending.pop(c):
                cp.wait()
            if c - 2 in out_cps:
                out_cps.pop(c - 2).wait()
            _pack_chunk(rows[c % 2], pk[c % 2])
            out_cps[c] = pltpu.async_copy(
                pk[c % 2],
                out_ref.at[pl.ds(pl.multiple_of((base + c * _CH) // 2, 8),
                                 _CH // 2)],
                so[c % 2])
        for cp in out_cps.values():
            cp.wait()

    return functools.partial(
        pl.kernel,
        mesh=plsc.VectorSubcoreMesh(core_axis_name="c", subcore_axis_name="s"),
        out_type=jax.ShapeDtypeStruct((n_rows // 2, _DP), jnp.float32),
        scratch_types=[
            pltpu.VMEM((rpw,), jnp.int32),
            pltpu.VMEM((_CH, _DP), jnp.float32),
            pltpu.VMEM((_CH, _DP), jnp.float32),
            pltpu.VMEM((_CH // 2, _DP), jnp.float32),
            pltpu.VMEM((_CH // 2, _DP), jnp.float32),
            pltpu.SemaphoreType.DMA,
            pltpu.SemaphoreType.DMA,
            pltpu.SemaphoreType.DMA,
            pltpu.SemaphoreType.DMA,
        ],
    )(_sc_gather_body)


def _score_call(xh, cbt):
    bh = xh.shape[0]
    return pl.pallas_call(
        _score_body,
        grid=(bh // _BLK,),
        in_specs=[
            pl.BlockSpec((_BLK, _NQ * _D), lambda i: (i, 0)),
            pl.BlockSpec((_NQ, _D, _K), lambda i: (0, 0, 0)),
        ],
        out_specs=[
            pl.BlockSpec((_BLK, _NQ), lambda i: (i, 0)),
            pl.BlockSpec((_BLK, _NQ), lambda i: (i, 0)),
        ],
        out_shape=[
            jax.ShapeDtypeStruct((bh, _NQ), jnp.int32),
            jax.ShapeDtypeStruct((bh, _NQ), jnp.int32),
        ],
    )(xh, cbt)


def kernel(x, codebooks):
    B = x.shape[0]
    half = B // 2
    cbt = codebooks.transpose(0, 2, 1)
    table = jnp.pad(codebooks.reshape(_NQ * _K, _D),
                    ((0, 0), (0, _DP - _D)))
    sc_gather = _make_sc_gather(half * _NQ)

    ids_a, gidx_a = _score_call(x[:half], cbt)
    ids_b, gidx_b = _score_call(x[half:], cbt)
    qa = sc_gather(table, gidx_a.reshape(half * _NQ))
    qb = sc_gather(table, gidx_b.reshape(half * _NQ))

    q_out = jnp.concatenate(
        [qa.reshape(half, _NQ * _D), qb.reshape(half, _NQ * _D)], axis=0)
    ids = jnp.concatenate([ids_a, ids_b], axis=0)
    return q_out, ids.T.astype(jnp.int64)
